# Initial kernel scaffold; baseline (speedup 1.0000x reference)
import jax
import jax.numpy as jnp
from jax.experimental import pallas as pl


def kernel(pos, pe):
    raise NotImplementedError("write your pallas kernel here")


# speedup vs baseline: 2.1695x; 2.1695x over previous
import functools
import jax
import jax.numpy as jnp
from jax import lax
from jax.experimental import pallas as pl
from jax.experimental.pallas import tpu as pltpu
from jax.experimental.pallas import tpu_sc as plsc

D_MODEL = 1024
MAX_LEN = 8192
N_ROWS = 4 * 8192
NC, NS, L = 2, 16, 16
NW = NC * NS
B_PER_W = N_ROWS // NW
CHUNK = 64
NCH = B_PER_W // CHUNK
VECS = B_PER_W // L

_MAGIC = 12582912.0


def _pe_lookup_body(pos_hbm, pe_hbm, out_hbm, pos_v, idx_v, rows_v, sem):
    wid = lax.axis_index("s") * NC + lax.axis_index("c")
    base = wid * B_PER_W

    pltpu.sync_copy(pos_hbm.at[pl.ds(base, B_PER_W)], pos_v)

    def idx_body(j, _):
        x = pos_v[pl.ds(j * L, L)]
        y = x * float(MAX_LEN)
        r = (y + _MAGIC) - _MAGIC
        r = jnp.minimum(jnp.maximum(r, 0.0), float(MAX_LEN - 1))
        g = j // (CHUNK // L)
        k = j % (CHUNK // L)
        idx_v[g, pl.ds(k * L, L)] = r.astype(jnp.int32)
        return 0

    lax.fori_loop(0, VECS, idx_body, 0)

    def chunk_body(g, _):
        pltpu.async_copy(pe_hbm.at[idx_v.at[g]], rows_v, sem).wait()
        pltpu.sync_copy(rows_v, out_hbm.at[pl.ds(base + g * CHUNK, CHUNK)])
        return 0

    lax.fori_loop(0, NCH, chunk_body, 0)


@jax.jit
def _pe_lookup(pos_flat, pe):
    mesh = plsc.VectorSubcoreMesh(core_axis_name="c", subcore_axis_name="s")
    run = pl.kernel(
        _pe_lookup_body,
        out_type=jax.ShapeDtypeStruct((N_ROWS, D_MODEL), jnp.float32),
        mesh=mesh,
        scratch_types=[
            pltpu.VMEM((B_PER_W,), jnp.float32),
            pltpu.VMEM((NCH, CHUNK), jnp.int32),
            pltpu.VMEM((CHUNK, D_MODEL), jnp.float32),
            pltpu.SemaphoreType.DMA,
        ],
    )
    return run(pos_flat, pe)


def kernel(pos, pe):
    out = _pe_lookup(pos.reshape(-1), pe)
    return out.reshape(pos.shape + (pe.shape[1],))


# speedup vs baseline: 2.3692x; 1.0920x over previous
import jax
import jax.numpy as jnp
from jax import lax
from jax.experimental import pallas as pl
from jax.experimental.pallas import tpu as pltpu
from jax.experimental.pallas import tpu_sc as plsc

D_MODEL = 1024
MAX_LEN = 8192
N_ROWS = 4 * 8192
NC, NS, L = 2, 16, 16
NW = NC * NS
B_PER_W = N_ROWS // NW
CHUNK = 32
NCH = B_PER_W // CHUNK
VECS = B_PER_W // L

_MAGIC = 12582912.0


def _pe_lookup_body(pos_hbm, pe_hbm, out_hbm, pos_v, idx_v,
                    rows0, rows1, gsem0, gsem1, wsem0, wsem1):
    wid = lax.axis_index("s") * NC + lax.axis_index("c")
    base = wid * B_PER_W
    rows = (rows0, rows1)
    gsem = (gsem0, gsem1)
    wsem = (wsem0, wsem1)

    pltpu.sync_copy(pos_hbm.at[pl.ds(base, B_PER_W)], pos_v)

    def idx_body(j, _):
        x = pos_v[pl.ds(j * L, L)]
        y = x * float(MAX_LEN)
        r = (y + _MAGIC) - _MAGIC
        r = jnp.minimum(jnp.maximum(r, 0.0), float(MAX_LEN - 1))
        g = j // (CHUNK // L)
        k = j % (CHUNK // L)
        idx_v[g, pl.ds(k * L, L)] = r.astype(jnp.int32)
        return 0

    lax.fori_loop(0, VECS, idx_body, 0)

    pltpu.async_copy(pe_hbm.at[idx_v.at[0]], rows0, gsem0)
    pltpu.async_copy(pe_hbm.at[idx_v.at[1]], rows1, gsem1)

    def outer(i, _):
        for b in range(2):
            g = 2 * i + b
            pltpu.make_async_copy(
                pe_hbm.at[idx_v.at[g]], rows[b], gsem[b]).wait()
            pltpu.async_copy(
                rows[b], out_hbm.at[pl.ds(base + g * CHUNK, CHUNK)], wsem[b])

            @pl.when(g + 2 < NCH)
            def _():
                pltpu.make_async_copy(
                    rows[b], out_hbm.at[pl.ds(base + g * CHUNK, CHUNK)],
                    wsem[b]).wait()
                pltpu.async_copy(pe_hbm.at[idx_v.at[g + 2]], rows[b], gsem[b])
        return 0

    lax.fori_loop(0, NCH // 2, outer, 0)

    for b, g in ((0, NCH - 2), (1, NCH - 1)):
        pltpu.make_async_copy(
            rows[b], out_hbm.at[pl.ds(base + g * CHUNK, CHUNK)],
            wsem[b]).wait()


@jax.jit
def _pe_lookup(pos_flat, pe):
    mesh = plsc.VectorSubcoreMesh(core_axis_name="c", subcore_axis_name="s")
    run = pl.kernel(
        _pe_lookup_body,
        out_type=jax.ShapeDtypeStruct((N_ROWS, D_MODEL), jnp.float32),
        mesh=mesh,
        scratch_types=[
            pltpu.VMEM((B_PER_W,), jnp.float32),
            pltpu.VMEM((NCH, CHUNK), jnp.int32),
            pltpu.VMEM((CHUNK, D_MODEL), jnp.float32),
            pltpu.VMEM((CHUNK, D_MODEL), jnp.float32),
            pltpu.SemaphoreType.DMA,
            pltpu.SemaphoreType.DMA,
            pltpu.SemaphoreType.DMA,
            pltpu.SemaphoreType.DMA,
        ],
    )
    return run(pos_flat, pe)


def kernel(pos, pe):
    out = _pe_lookup(pos.reshape(-1), pe)
    return out.reshape(pos.shape + (pe.shape[1],))
